# baseline (speedup 1.0000x reference)
import functools

import jax
import jax.numpy as jnp
from jax import lax
from jax.experimental import pallas as pl
from jax.experimental.pallas import tpu as pltpu
from jax.experimental.pallas import tpu_sc as plsc

_N_USERS = 100_000
_N_ITEMS = 1_000_000
_RANK = 32
_BLOCK = 65536
_GRID = (_N_ITEMS + _BLOCK - 1) // _BLOCK

_NW = 32
_TILES_PER_W = 240
_CHUNK_T = 30
_N_CH = _TILES_PER_W // _CHUNK_T


def _sc_stream_body(vt_ref, out_ref, b0, b1, zbuf, s0, s1):
    wid = lax.axis_index("s") * 2 + lax.axis_index("c")
    base_t = wid * _TILES_PER_W

    def src(t):
        return vt_ref.at[pl.ds(0, 8), pl.ds((base_t + t) * 128, _CHUNK_T * 128)]

    pltpu.make_async_copy(src(0), b0, s0).start()
    pltpu.make_async_copy(src(_CHUNK_T), b1, s1).start()

    def step(j, carry):
        t0 = (2 * j + 2) * _CHUNK_T
        pltpu.make_async_copy(src(t0), b0, s0).wait()
        pltpu.make_async_copy(src(t0), b0, s0).start()
        t1 = (2 * j + 3) * _CHUNK_T
        pltpu.make_async_copy(src(t1), b1, s1).wait()
        pltpu.make_async_copy(src(t1), b1, s1).start()
        return carry

    lax.fori_loop(0, _N_CH // 2 - 1, step, 0)
    pltpu.make_async_copy(src(0), b0, s0).wait()
    pltpu.make_async_copy(src(0), b1, s1).wait()

    @pl.when(wid == 0)
    def _write_out():
        zbuf[pl.ds(0, 16)] = jnp.zeros((16,), jnp.float32)
        pltpu.sync_copy(zbuf, out_ref)


def _sc_stream(vt):
    mesh = plsc.VectorSubcoreMesh(core_axis_name="c", subcore_axis_name="s")
    kern = functools.partial(
        pl.kernel,
        mesh=mesh,
        out_type=jax.ShapeDtypeStruct((16,), jnp.float32),
        scratch_types=[
            pltpu.VMEM((8, _CHUNK_T * 128), jnp.float32),
            pltpu.VMEM((8, _CHUNK_T * 128), jnp.float32),
            pltpu.VMEM((16,), jnp.float32),
            pltpu.SemaphoreType.DMA,
            pltpu.SemaphoreType.DMA,
        ],
        compiler_params=pltpu.CompilerParams(
            has_side_effects=True,
            use_tc_tiling_on_sc=True,
        ),
    )(_sc_stream_body)
    return kern(vt)


def _score_body(uid_ref, ub_ref, vt_ref, out_ref):
    c = uid_ref[0] % 128
    lane = jax.lax.broadcasted_iota(jnp.int32, (_RANK, 128), 1)
    u_col = jnp.sum(
        jnp.where(lane == c, ub_ref[...], 0.0), axis=1, keepdims=True
    )
    scores = jax.lax.dot_general(
        u_col,
        vt_ref[...],
        dimension_numbers=(((0,), (0,)), ((), ())),
        preferred_element_type=jnp.float32,
    )
    out_ref[...] = scores.reshape((_BLOCK,))


def kernel(user_id, U, V):
    uid = jnp.asarray(user_id, jnp.int32).reshape((1,))
    ut = U.T
    vt = V.T
    _ = _sc_stream(vt)
    grid_spec = pltpu.PrefetchScalarGridSpec(
        num_scalar_prefetch=1,
        grid=(_GRID,),
        in_specs=[
            pl.BlockSpec((_RANK, 128), lambda i, uid_ref: (0, uid_ref[0] // 128)),
            pl.BlockSpec((_RANK, _BLOCK), lambda i, uid_ref: (0, i)),
        ],
        out_specs=pl.BlockSpec((_BLOCK,), lambda i, uid_ref: (i,)),
    )
    return pl.pallas_call(
        _score_body,
        grid_spec=grid_spec,
        out_shape=jax.ShapeDtypeStruct((_N_ITEMS,), jnp.float32),
    )(uid, ut, vt)


# speedup vs baseline: 1.6190x; 1.6190x over previous
import jax
import jax.numpy as jnp
from jax.experimental import pallas as pl
from jax.experimental.pallas import tpu as pltpu

_N_USERS = 100_000
_N_ITEMS = 1_000_000
_RANK = 32
_BLOCK = 65536
_GRID = (_N_ITEMS + _BLOCK - 1) // _BLOCK


def _score_body(uid_ref, ub_ref, vt_ref, out_ref):
    c = uid_ref[0] % 128
    lane = jax.lax.broadcasted_iota(jnp.int32, (_RANK, 128), 1)
    u_col = jnp.sum(
        jnp.where(lane == c, ub_ref[...], 0.0), axis=1, keepdims=True
    )
    scores = jax.lax.dot_general(
        u_col,
        vt_ref[...],
        dimension_numbers=(((0,), (0,)), ((), ())),
        preferred_element_type=jnp.float32,
    )
    out_ref[...] = scores.reshape((_BLOCK,))


def kernel(user_id, U, V):
    uid = jnp.asarray(user_id, jnp.int32).reshape((1,))
    ut = U.T
    vt = V.T
    grid_spec = pltpu.PrefetchScalarGridSpec(
        num_scalar_prefetch=1,
        grid=(_GRID,),
        in_specs=[
            pl.BlockSpec((_RANK, 128), lambda i, uid_ref: (0, uid_ref[0] // 128)),
            pl.BlockSpec((_RANK, _BLOCK), lambda i, uid_ref: (0, i)),
        ],
        out_specs=pl.BlockSpec((_BLOCK,), lambda i, uid_ref: (i,)),
    )
    return pl.pallas_call(
        _score_body,
        grid_spec=grid_spec,
        out_shape=jax.ShapeDtypeStruct((_N_ITEMS,), jnp.float32),
    )(uid, ut, vt)
